# baseline (speedup 1.0000x reference)
import functools

import jax
import jax.numpy as jnp
from jax import lax
from jax.experimental import pallas as pl
from jax.experimental.pallas import tpu as pltpu
from jax.experimental.pallas import tpu_sc as plsc

_N = 32768
_BINS = 16
_NS = 16
_CHUNK = _N // _NS
_VECS = _CHUNK // 16
_ROWSTRIDE = 64


def _hist_body(ids_hbm, w_hbm, par_hbm, out_hbm,
               ids_v, w_v, par_v, hist_v, comb_v, shared,
               sem0, sem1, sem2):
    sid = lax.axis_index("s")
    base = sid * _CHUNK

    cp0 = pltpu.async_copy(ids_hbm.at[pl.ds(base, _CHUNK)], ids_v, sem0)
    cp1 = pltpu.async_copy(w_hbm.at[pl.ds(base, _CHUNK)], w_v, sem1)
    cp2 = pltpu.async_copy(par_hbm, par_v, sem2)
    cp0.wait()
    cp1.wait()
    cp2.wait()

    lane = lax.iota(jnp.int32, 16)
    nt = par_v[pl.ds(0, 16)]
    zero = jnp.zeros((16,), jnp.float32)

    def step(j, accs):
        off = j * 16
        ids = ids_v[pl.ds(off, 16)]
        w = w_v[pl.ds(off, 16)]
        pos = lane + (base + off)
        w = jnp.where(pos < nt, w, zero)
        return tuple(accs[b] + jnp.where(ids == b, w, zero)
                     for b in range(_BINS))

    accs = lax.fori_loop(0, _VECS, step,
                         tuple(zero for _ in range(_BINS)))

    tile_hist = zero
    for b in range(_BINS):
        row = accs[b]
        s = row[0]
        for j in range(1, 16):
            s = s + row[j]
        tile_hist = tile_hist + jnp.where(lane == b, s, 0.0)
    hist_v[...] = tile_hist

    pltpu.sync_copy(hist_v, shared.at[pl.ds(sid * _ROWSTRIDE, 16)])
    plsc.subcore_barrier()

    @pl.when(sid == 0)
    def _():
        pltpu.sync_copy(shared, comb_v)
        tot = zero
        for i in range(_NS):
            tot = tot + comb_v[pl.ds(i * _ROWSTRIDE, 16)]
        ms = par_v[pl.ds(16, 16)]
        tot = jnp.where(lane < ms, tot, zero)
        hist_v[...] = tot
        pltpu.sync_copy(hist_v, out_hbm)


@functools.partial(
    pl.kernel,
    mesh=plsc.VectorSubcoreMesh(core_axis_name="c", subcore_axis_name="s",
                                num_cores=1),
    out_type=jax.ShapeDtypeStruct((_BINS,), jnp.float32),
    scratch_types=[
        pltpu.VMEM((_CHUNK,), jnp.int32),
        pltpu.VMEM((_CHUNK,), jnp.float32),
        pltpu.VMEM((32,), jnp.int32),
        pltpu.VMEM((16,), jnp.float32),
        pltpu.VMEM((_NS * _ROWSTRIDE,), jnp.float32),
        pltpu.VMEM_SHARED((_NS * _ROWSTRIDE,), jnp.float32),
        pltpu.SemaphoreType.DMA,
        pltpu.SemaphoreType.DMA,
        pltpu.SemaphoreType.DMA,
    ],
)
def _hist_kernel(ids_hbm, w_hbm, par_hbm, out_hbm,
                 ids_v, w_v, par_v, hist_v, comb_v, shared,
                 sem0, sem1, sem2):
    _hist_body(ids_hbm, w_hbm, par_hbm, out_hbm,
               ids_v, w_v, par_v, hist_v, comb_v, shared,
               sem0, sem1, sem2)


def kernel(tokens, seq_ids, pos_ids, weights, num_tokens, max_sequences):
    par = jnp.concatenate([
        jnp.full((16,), num_tokens, dtype=jnp.int32),
        jnp.full((16,), max_sequences, dtype=jnp.int32),
    ])
    return _hist_kernel(seq_ids, weights, par)


# speedup vs baseline: 1.0148x; 1.0148x over previous
import functools

import jax
import jax.numpy as jnp
from jax import lax
from jax.experimental import pallas as pl
from jax.experimental.pallas import tpu as pltpu
from jax.experimental.pallas import tpu_sc as plsc

_N = 32768
_BINS = 16
_NS = 16
_CHUNK = _N // _NS
_VECS = _CHUNK // 16
_ROWSTRIDE = 64


def _hist_body(ids_hbm, w_hbm, par_hbm, out_hbm,
               ids_v, w_v, par_v, hist_v, fold_v, comb_v, shared,
               sem0, sem1, sem2):
    sid = lax.axis_index("s")
    base = sid * _CHUNK

    cp0 = pltpu.async_copy(ids_hbm.at[pl.ds(base, _CHUNK)], ids_v, sem0)
    cp1 = pltpu.async_copy(w_hbm.at[pl.ds(base, _CHUNK)], w_v, sem1)
    cp2 = pltpu.async_copy(par_hbm, par_v, sem2)
    cp0.wait()
    cp1.wait()
    cp2.wait()

    lane = lax.iota(jnp.int32, 16)
    nt = par_v[pl.ds(0, 16)]
    zero = jnp.zeros((16,), jnp.float32)

    def step(j, accs):
        off = j * 16
        ids = ids_v[pl.ds(off, 16)]
        w = w_v[pl.ds(off, 16)]
        pos = lane + (base + off)
        w = jnp.where(pos < nt, w, zero)
        return tuple(accs[b] + jnp.where(ids == b, w, zero)
                     for b in range(_BINS))

    accs = lax.fori_loop(0, _VECS, step,
                         tuple(zero for _ in range(_BINS)))

    tile_hist = zero
    for b in range(_BINS):
        cur = accs[b]
        off = b * 32
        for r in (8, 4, 2, 1):
            fold_v[pl.ds(off, 16)] = cur
            cur = cur + fold_v[pl.ds(off + r, 16)]
        tile_hist = tile_hist + jnp.where(lane == b, cur[0], 0.0)
    hist_v[...] = tile_hist

    pltpu.sync_copy(hist_v, shared.at[pl.ds(sid * _ROWSTRIDE, 16)])
    plsc.subcore_barrier()

    @pl.when(sid == 0)
    def _():
        pltpu.sync_copy(shared, comb_v)
        tot = zero
        for i in range(_NS):
            tot = tot + comb_v[pl.ds(i * _ROWSTRIDE, 16)]
        ms = par_v[pl.ds(16, 16)]
        tot = jnp.where(lane < ms, tot, zero)
        hist_v[...] = tot
        pltpu.sync_copy(hist_v, out_hbm)


@functools.partial(
    pl.kernel,
    mesh=plsc.VectorSubcoreMesh(core_axis_name="c", subcore_axis_name="s",
                                num_cores=1),
    out_type=jax.ShapeDtypeStruct((_BINS,), jnp.float32),
    scratch_types=[
        pltpu.VMEM((_CHUNK,), jnp.int32),
        pltpu.VMEM((_CHUNK,), jnp.float32),
        pltpu.VMEM((32,), jnp.int32),
        pltpu.VMEM((16,), jnp.float32),
        pltpu.VMEM((_BINS * 32,), jnp.float32),
        pltpu.VMEM((_NS * _ROWSTRIDE,), jnp.float32),
        pltpu.VMEM_SHARED((_NS * _ROWSTRIDE,), jnp.float32),
        pltpu.SemaphoreType.DMA,
        pltpu.SemaphoreType.DMA,
        pltpu.SemaphoreType.DMA,
    ],
)
def _hist_kernel(ids_hbm, w_hbm, par_hbm, out_hbm,
                 ids_v, w_v, par_v, hist_v, fold_v, comb_v, shared,
                 sem0, sem1, sem2):
    _hist_body(ids_hbm, w_hbm, par_hbm, out_hbm,
               ids_v, w_v, par_v, hist_v, fold_v, comb_v, shared,
               sem0, sem1, sem2)


def kernel(tokens, seq_ids, pos_ids, weights, num_tokens, max_sequences):
    par = jnp.concatenate([
        jnp.full((16,), num_tokens, dtype=jnp.int32),
        jnp.full((16,), max_sequences, dtype=jnp.int32),
    ])
    return _hist_kernel(seq_ids, weights, par)


# speedup vs baseline: 1.0295x; 1.0144x over previous
import functools

import jax
import jax.numpy as jnp
from jax import lax
from jax.experimental import pallas as pl
from jax.experimental.pallas import tpu as pltpu
from jax.experimental.pallas import tpu_sc as plsc

_N = 32768
_BINS = 16
_NS = 16
_CHUNK = _N // _NS
_VECS = _CHUNK // 16
_ROWSTRIDE = 64


def _hist_body(ids_hbm, w_hbm, par_hbm, out_hbm,
               ids_v, w_v, par_v, hist_v, fold_v, comb_v, shared,
               sem0, sem1, sem2):
    sid = lax.axis_index("s")
    base = sid * _CHUNK

    cp0 = pltpu.async_copy(ids_hbm.at[pl.ds(base, _CHUNK)], ids_v, sem0)
    cp1 = pltpu.async_copy(w_hbm.at[pl.ds(base, _CHUNK)], w_v, sem1)
    cp2 = pltpu.async_copy(par_hbm, par_v, sem2)
    cp0.wait()
    cp1.wait()
    cp2.wait()

    lane = lax.iota(jnp.int32, 16)
    nt = par_v[pl.ds(0, 16)][0]
    zero = jnp.zeros((16,), jnp.float32)

    n_valid = jnp.clip(nt - base, 0, _CHUNK)
    n_full = n_valid // 16
    rem = n_valid - n_full * 16

    def step(j, accs):
        off = j * 16
        ids = ids_v[pl.ds(off, 16)]
        w = w_v[pl.ds(off, 16)]
        return tuple(jnp.where(ids == b, accs[b] + w, accs[b])
                     for b in range(_BINS))

    accs = lax.fori_loop(0, n_full, step,
                         tuple(zero for _ in range(_BINS)))

    offp = jnp.minimum(n_full, _VECS - 1) * 16
    ids_p = ids_v[pl.ds(offp, 16)]
    w_p = jnp.where(lane < rem, w_v[pl.ds(offp, 16)], zero)
    accs = tuple(jnp.where(ids_p == b, accs[b] + w_p, accs[b])
                 for b in range(_BINS))

    tile_hist = zero
    for b in range(_BINS):
        cur = accs[b]
        off = b * 32
        for r in (8, 4, 2, 1):
            fold_v[pl.ds(off, 16)] = cur
            cur = cur + fold_v[pl.ds(off + r, 16)]
        tile_hist = tile_hist + jnp.where(lane == b, cur[0], 0.0)
    hist_v[...] = tile_hist

    pltpu.sync_copy(hist_v, shared.at[pl.ds(sid * _ROWSTRIDE, 16)])
    plsc.subcore_barrier()

    @pl.when(sid == 0)
    def _():
        pltpu.sync_copy(shared, comb_v)
        tot = zero
        for i in range(_NS):
            tot = tot + comb_v[pl.ds(i * _ROWSTRIDE, 16)]
        ms = par_v[pl.ds(16, 16)]
        tot = jnp.where(lane < ms, tot, zero)
        hist_v[...] = tot
        pltpu.sync_copy(hist_v, out_hbm)


@functools.partial(
    pl.kernel,
    mesh=plsc.VectorSubcoreMesh(core_axis_name="c", subcore_axis_name="s",
                                num_cores=1),
    out_type=jax.ShapeDtypeStruct((_BINS,), jnp.float32),
    scratch_types=[
        pltpu.VMEM((_CHUNK,), jnp.int32),
        pltpu.VMEM((_CHUNK,), jnp.float32),
        pltpu.VMEM((32,), jnp.int32),
        pltpu.VMEM((16,), jnp.float32),
        pltpu.VMEM((_BINS * 32,), jnp.float32),
        pltpu.VMEM((_NS * _ROWSTRIDE,), jnp.float32),
        pltpu.VMEM_SHARED((_NS * _ROWSTRIDE,), jnp.float32),
        pltpu.SemaphoreType.DMA,
        pltpu.SemaphoreType.DMA,
        pltpu.SemaphoreType.DMA,
    ],
)
def _hist_kernel(ids_hbm, w_hbm, par_hbm, out_hbm,
                 ids_v, w_v, par_v, hist_v, fold_v, comb_v, shared,
                 sem0, sem1, sem2):
    _hist_body(ids_hbm, w_hbm, par_hbm, out_hbm,
               ids_v, w_v, par_v, hist_v, fold_v, comb_v, shared,
               sem0, sem1, sem2)


def kernel(tokens, seq_ids, pos_ids, weights, num_tokens, max_sequences):
    par = jnp.concatenate([
        jnp.full((16,), num_tokens, dtype=jnp.int32),
        jnp.full((16,), max_sequences, dtype=jnp.int32),
    ])
    return _hist_kernel(seq_ids, weights, par)
